# Initial kernel scaffold; baseline (speedup 1.0000x reference)
import functools

import jax
import jax.numpy as jnp
from jax import lax
from jax.experimental import pallas as pl
from jax.experimental.pallas import tpu as pltpu
from jax.experimental.pallas import tpu_sc as plsc

_NC = 2
_NS = 16
_NW = _NC * _NS
_L = 16


def _prep_ab(x, W_msg):
    N, D = x.shape
    blk = 2000
    assert N % blk == 0

    def body(x_ref, w_ref, a_ref, b_ref):
        xb = x_ref[...]
        a_ref[...] = jnp.dot(xb, w_ref[0:D, :], preferred_element_type=jnp.float32)
        b_ref[...] = jnp.dot(xb, w_ref[D:2 * D, :], preferred_element_type=jnp.float32)

    return pl.pallas_call(
        body,
        grid=(N // blk,),
        in_specs=[
            pl.BlockSpec((blk, D), lambda i: (i, 0)),
            pl.BlockSpec(W_msg.shape, lambda i: (0, 0)),
        ],
        out_specs=[
            pl.BlockSpec((blk, D), lambda i: (i, 0)),
            pl.BlockSpec((blk, D), lambda i: (i, 0)),
        ],
        out_shape=[jax.ShapeDtypeStruct((N, D), jnp.float32)] * 2,
    )(x, W_msg)


def _prep_c(edge_attr, W_msg, b_msg, D):
    E, DE = edge_attr.shape
    blk = 2000
    assert E % blk == 0

    def body(e_ref, w_ref, b_ref, o_ref):
        o_ref[...] = (
            jnp.dot(e_ref[...], w_ref[2 * D:2 * D + DE, :],
                    preferred_element_type=jnp.float32)
            + b_ref[...]
        )

    return pl.pallas_call(
        body,
        grid=(E // blk,),
        in_specs=[
            pl.BlockSpec((blk, DE), lambda i: (i, 0)),
            pl.BlockSpec(W_msg.shape, lambda i: (0, 0)),
            pl.BlockSpec((1, D), lambda i: (0, 0)),
        ],
        out_specs=pl.BlockSpec((blk, D), lambda i: (i, 0)),
        out_shape=jax.ShapeDtypeStruct((E, D), jnp.float32),
    )(edge_attr, W_msg, b_msg.reshape(1, D))


def _edge_sc(A, B, C, src, dst):
    N, D = A.shape
    E = C.shape[0]
    assert E % _NW == 0
    EW = E // _NW
    assert EW % 8 == 0
    K = 128
    FULL = EW // K
    REM = EW - FULL * K
    ROWS_T = N // _NS
    ZR = 125
    assert ROWS_T % ZR == 0
    NZ = ROWS_T // ZR

    mesh = plsc.VectorSubcoreMesh(core_axis_name="c", subcore_axis_name="s")
    scratch = [
        pltpu.VMEM((K,), jnp.int32),
        pltpu.VMEM((K,), jnp.int32),
        pltpu.VMEM((K, D), jnp.float32),
        pltpu.VMEM((K, D), jnp.float32),
        pltpu.VMEM((K, D), jnp.float32),
        pltpu.VMEM((max(REM, 8),), jnp.int32),
        pltpu.VMEM((max(REM, 8),), jnp.int32),
        pltpu.VMEM((max(REM, 1), D), jnp.float32),
        pltpu.VMEM((max(REM, 1), D), jnp.float32),
        pltpu.VMEM((max(REM, 1), D), jnp.float32),
        pltpu.VMEM((ZR, D), jnp.float32),
        pltpu.VMEM_SHARED((N, D), jnp.float32),
        pltpu.SemaphoreType.DMA,
        pltpu.SemaphoreType.DMA,
        pltpu.SemaphoreType.DMA,
    ]

    @functools.partial(
        pl.kernel,
        out_type=jax.ShapeDtypeStruct((_NC, N, D), jnp.float32),
        mesh=mesh,
        scratch_types=scratch,
    )
    def body(a_hbm, b_hbm, c_hbm, src_hbm, dst_hbm, out_hbm,
             srcv, dstv, av, bv, cv, srcr, dstr, ar, br, cr,
             zbuf, agg, sem_a, sem_b, sem_c):
        cid = lax.axis_index("c")
        sid = lax.axis_index("s")
        wid = cid * _NS + sid

        zero = jnp.zeros((_L,), jnp.float32)

        def zrow(r, carry):
            for jj in range(D // _L):
                zbuf[r, pl.ds(jj * _L, _L)] = zero
            return carry

        lax.fori_loop(0, ZR, zrow, 0)

        def zslice(k, carry):
            pltpu.sync_copy(zbuf, agg.at[pl.ds(sid * ROWS_T + k * ZR, ZR)])
            return carry

        lax.fori_loop(0, NZ, zslice, 0)
        plsc.subcore_barrier()

        base = wid * EW

        def do_chunk(off, s_i, d_i, a_b, b_b, c_b, n):
            pltpu.sync_copy(src_hbm.at[pl.ds(off, n)], s_i)
            pltpu.sync_copy(dst_hbm.at[pl.ds(off, n)], d_i)
            ca = pltpu.async_copy(a_hbm.at[s_i], a_b, sem_a)
            cb = pltpu.async_copy(b_hbm.at[d_i], b_b, sem_b)
            cc = pltpu.async_copy(c_hbm.at[pl.ds(off, n)], c_b, sem_c)
            ca.wait()
            cb.wait()
            cc.wait()

            def crow(r, carry):
                for jj in range(D // _L):
                    sl = pl.ds(jj * _L, _L)
                    a_b[r, sl] = jnp.maximum(a_b[r, sl] + b_b[r, sl] + c_b[r, sl], 0.0)
                return carry

            lax.fori_loop(0, n, crow, 0)
            pltpu.sync_copy(a_b, agg.at[d_i], add=True)

        def chunk_loop(j, carry):
            do_chunk(base + j * K, srcv, dstv, av, bv, cv, K)
            return carry

        lax.fori_loop(0, FULL, chunk_loop, 0)
        if REM:
            do_chunk(base + FULL * K, srcr, dstr, ar, br, cr, REM)

        plsc.subcore_barrier()

        def orow(k, carry):
            r0 = sid * ROWS_T + k * ZR
            pltpu.sync_copy(agg.at[pl.ds(r0, ZR)], zbuf)
            pltpu.sync_copy(zbuf, out_hbm.at[cid, pl.ds(r0, ZR)])
            return carry

        lax.fori_loop(0, NZ, orow, 0)

    return body(A, B, C, src, dst)


def _update(x, agg2, W_upd, b_upd):
    N, D = x.shape
    blk = 2000
    assert N % blk == 0

    def body(x_ref, g_ref, w_ref, b_ref, o_ref):
        xb = x_ref[...]
        g = g_ref[0] + g_ref[1]
        h = (
            jnp.dot(xb, w_ref[0:D, :], preferred_element_type=jnp.float32)
            + jnp.dot(g, w_ref[D:2 * D, :], preferred_element_type=jnp.float32)
            + b_ref[...]
        )
        o_ref[...] = xb + jnp.maximum(h, 0.0)

    return pl.pallas_call(
        body,
        grid=(N // blk,),
        in_specs=[
            pl.BlockSpec((blk, D), lambda i: (i, 0)),
            pl.BlockSpec((2, blk, D), lambda i: (0, i, 0)),
            pl.BlockSpec(W_upd.shape, lambda i: (0, 0)),
            pl.BlockSpec((1, D), lambda i: (0, 0)),
        ],
        out_specs=pl.BlockSpec((blk, D), lambda i: (i, 0)),
        out_shape=jax.ShapeDtypeStruct((N, D), jnp.float32),
    )(x, agg2, W_upd, b_upd.reshape(1, D))


def kernel(x, edge_index, edge_attr, W_msg, b_msg, W_upd, b_upd):
    N, D = x.shape
    src = edge_index[0].astype(jnp.int32)
    dst = edge_index[1].astype(jnp.int32)
    A, B = _prep_ab(x, W_msg)
    C = _prep_c(edge_attr, W_msg, b_msg, D)
    agg2 = _edge_sc(A, B, C, src, dst)
    return _update(x, agg2, W_upd, b_upd)


# speedup vs baseline: 3.6220x; 3.6220x over previous
import functools

import jax
import jax.numpy as jnp
from jax import lax
from jax.experimental import pallas as pl
from jax.experimental.pallas import tpu as pltpu
from jax.experimental.pallas import tpu_sc as plsc

_NC = 2
_NS = 16
_NW = _NC * _NS
_L = 16


def _prep_ab(x, W_msg):
    N, D = x.shape
    blk = 2000
    assert N % blk == 0

    def body(x_ref, w_ref, a_ref, b_ref):
        xb = x_ref[...]
        a_ref[...] = jnp.dot(xb, w_ref[0:D, :], preferred_element_type=jnp.float32)
        b_ref[...] = jnp.dot(xb, w_ref[D:2 * D, :], preferred_element_type=jnp.float32)

    return pl.pallas_call(
        body,
        grid=(N // blk,),
        in_specs=[
            pl.BlockSpec((blk, D), lambda i: (i, 0)),
            pl.BlockSpec(W_msg.shape, lambda i: (0, 0)),
        ],
        out_specs=[
            pl.BlockSpec((blk, D), lambda i: (i, 0)),
            pl.BlockSpec((blk, D), lambda i: (i, 0)),
        ],
        out_shape=[jax.ShapeDtypeStruct((N, D), jnp.float32)] * 2,
    )(x, W_msg)


def _prep_c(edge_attr, W_msg, b_msg, D):
    E, DE = edge_attr.shape
    blk = 2000
    assert E % blk == 0

    def body(e_ref, w_ref, b_ref, o_ref):
        o_ref[...] = (
            jnp.dot(e_ref[...], w_ref[2 * D:2 * D + DE, :],
                    preferred_element_type=jnp.float32)
            + b_ref[...]
        )

    return pl.pallas_call(
        body,
        grid=(E // blk,),
        in_specs=[
            pl.BlockSpec((blk, DE), lambda i: (i, 0)),
            pl.BlockSpec(W_msg.shape, lambda i: (0, 0)),
            pl.BlockSpec((1, D), lambda i: (0, 0)),
        ],
        out_specs=pl.BlockSpec((blk, D), lambda i: (i, 0)),
        out_shape=jax.ShapeDtypeStruct((E, D), jnp.float32),
    )(edge_attr, W_msg, b_msg.reshape(1, D))


def _edge_sc(A, B, C, src, dst):
    N, D = A.shape
    E = C.shape[0]
    assert E % _NW == 0
    EW = E // _NW
    assert EW % 8 == 0
    K = 96
    FULL = EW // K
    REM = EW - FULL * K
    assert REM % 8 == 0
    ROWS_T = (N // _NS) // 8 * 8
    TAIL = N - _NS * ROWS_T
    NZ = ROWS_T // K
    ZREM = ROWS_T - NZ * K
    assert ZREM % 8 == 0 and TAIL % 8 == 0

    mesh = plsc.VectorSubcoreMesh(core_axis_name="c", subcore_axis_name="s")
    scratch = [
        pltpu.VMEM((K,), jnp.int32),
        pltpu.VMEM((K,), jnp.int32),
        pltpu.VMEM((K, D), jnp.float32),
        pltpu.VMEM((K, D), jnp.float32),
        pltpu.VMEM((K, D), jnp.float32),
        pltpu.VMEM((max(REM, 8),), jnp.int32),
        pltpu.VMEM((max(REM, 8),), jnp.int32),
        pltpu.VMEM_SHARED((N, D), jnp.float32),
        pltpu.SemaphoreType.DMA,
        pltpu.SemaphoreType.DMA,
        pltpu.SemaphoreType.DMA,
    ]

    @functools.partial(
        pl.kernel,
        out_type=jax.ShapeDtypeStruct((_NC, N, D), jnp.float32),
        mesh=mesh,
        scratch_types=scratch,
    )
    def body(a_hbm, b_hbm, c_hbm, src_hbm, dst_hbm, out_hbm,
             srcv, dstv, av, bv, cv, srcr, dstr,
             agg, sem_a, sem_b, sem_c):
        cid = lax.axis_index("c")
        sid = lax.axis_index("s")
        wid = cid * _NS + sid

        zero = jnp.zeros((_L,), jnp.float32)

        def zrow(r, carry):
            for jj in range(D // _L):
                av[r, pl.ds(jj * _L, _L)] = zero
            return carry

        lax.fori_loop(0, K, zrow, 0)

        def zslice(k, carry):
            pltpu.sync_copy(av, agg.at[pl.ds(sid * ROWS_T + k * K, K)])
            return carry

        lax.fori_loop(0, NZ, zslice, 0)
        if ZREM:
            pltpu.sync_copy(av.at[pl.ds(0, ZREM)],
                            agg.at[pl.ds(sid * ROWS_T + NZ * K, ZREM)])

        @pl.when(sid == 0)
        def _zero_tail():
            pltpu.sync_copy(av.at[pl.ds(0, TAIL)],
                            agg.at[pl.ds(_NS * ROWS_T, TAIL)])

        plsc.subcore_barrier()

        base = wid * EW

        def do_chunk(off, s_i, d_i, n):
            pltpu.sync_copy(src_hbm.at[pl.ds(off, n)], s_i)
            pltpu.sync_copy(dst_hbm.at[pl.ds(off, n)], d_i)
            a_b = av.at[pl.ds(0, n)] if n != K else av
            b_b = bv.at[pl.ds(0, n)] if n != K else bv
            c_b = cv.at[pl.ds(0, n)] if n != K else cv
            ca = pltpu.async_copy(a_hbm.at[s_i], a_b, sem_a)
            cb = pltpu.async_copy(b_hbm.at[d_i], b_b, sem_b)
            cc = pltpu.async_copy(c_hbm.at[pl.ds(off, n)], c_b, sem_c)
            ca.wait()
            cb.wait()
            cc.wait()

            def crow(r, carry):
                for jj in range(D // _L):
                    sl = pl.ds(jj * _L, _L)
                    av[r, sl] = jnp.maximum(av[r, sl] + bv[r, sl] + cv[r, sl], 0.0)
                return carry

            lax.fori_loop(0, n, crow, 0)
            pltpu.sync_copy(a_b, agg.at[d_i], add=True)

        def chunk_loop(j, carry):
            do_chunk(base + j * K, srcv, dstv, K)
            return carry

        lax.fori_loop(0, FULL, chunk_loop, 0)
        if REM:
            do_chunk(base + FULL * K, srcr, dstr, REM)

        plsc.subcore_barrier()

        def orow(k, carry):
            r0 = sid * ROWS_T + k * K
            pltpu.sync_copy(agg.at[pl.ds(r0, K)], av)
            pltpu.sync_copy(av, out_hbm.at[cid, pl.ds(r0, K)])
            return carry

        lax.fori_loop(0, NZ, orow, 0)
        if ZREM:
            r0 = sid * ROWS_T + NZ * K
            pltpu.sync_copy(agg.at[pl.ds(r0, ZREM)], av.at[pl.ds(0, ZREM)])
            pltpu.sync_copy(av.at[pl.ds(0, ZREM)],
                            out_hbm.at[cid, pl.ds(r0, ZREM)])

        @pl.when(sid == 0)
        def _out_tail():
            r0 = _NS * ROWS_T
            pltpu.sync_copy(agg.at[pl.ds(r0, TAIL)], av.at[pl.ds(0, TAIL)])
            pltpu.sync_copy(av.at[pl.ds(0, TAIL)],
                            out_hbm.at[cid, pl.ds(r0, TAIL)])

    return body(A, B, C, src, dst)


def _update(x, agg2, W_upd, b_upd):
    N, D = x.shape
    blk = 2000
    assert N % blk == 0

    def body(x_ref, g_ref, w_ref, b_ref, o_ref):
        xb = x_ref[...]
        g = g_ref[0] + g_ref[1]
        h = (
            jnp.dot(xb, w_ref[0:D, :], preferred_element_type=jnp.float32)
            + jnp.dot(g, w_ref[D:2 * D, :], preferred_element_type=jnp.float32)
            + b_ref[...]
        )
        o_ref[...] = xb + jnp.maximum(h, 0.0)

    return pl.pallas_call(
        body,
        grid=(N // blk,),
        in_specs=[
            pl.BlockSpec((blk, D), lambda i: (i, 0)),
            pl.BlockSpec((2, blk, D), lambda i: (0, i, 0)),
            pl.BlockSpec(W_upd.shape, lambda i: (0, 0)),
            pl.BlockSpec((1, D), lambda i: (0, 0)),
        ],
        out_specs=pl.BlockSpec((blk, D), lambda i: (i, 0)),
        out_shape=jax.ShapeDtypeStruct((N, D), jnp.float32),
    )(x, agg2, W_upd, b_upd.reshape(1, D))


def kernel(x, edge_index, edge_attr, W_msg, b_msg, W_upd, b_upd):
    N, D = x.shape
    src = edge_index[0].astype(jnp.int32)
    dst = edge_index[1].astype(jnp.int32)
    A, B = _prep_ab(x, W_msg)
    C = _prep_c(edge_attr, W_msg, b_msg, D)
    agg2 = _edge_sc(A, B, C, src, dst)
    return _update(x, agg2, W_upd, b_upd)


# speedup vs baseline: 4.1702x; 1.1514x over previous
import functools

import jax
import jax.numpy as jnp
from jax import lax
from jax.experimental import pallas as pl
from jax.experimental.pallas import tpu as pltpu
from jax.experimental.pallas import tpu_sc as plsc

_NC = 2
_NS = 16
_NW = _NC * _NS
_L = 16


def _prep_ab(x, W_msg):
    N, D = x.shape
    blk = 2000
    assert N % blk == 0

    def body(x_ref, w_ref, a_ref, b_ref):
        xb = x_ref[...]
        a_ref[...] = jnp.dot(xb, w_ref[0:D, :], preferred_element_type=jnp.float32)
        b_ref[...] = jnp.dot(xb, w_ref[D:2 * D, :], preferred_element_type=jnp.float32)

    return pl.pallas_call(
        body,
        grid=(N // blk,),
        in_specs=[
            pl.BlockSpec((blk, D), lambda i: (i, 0)),
            pl.BlockSpec(W_msg.shape, lambda i: (0, 0)),
        ],
        out_specs=[
            pl.BlockSpec((blk, D), lambda i: (i, 0)),
            pl.BlockSpec((blk, D), lambda i: (i, 0)),
        ],
        out_shape=[jax.ShapeDtypeStruct((N, D), jnp.float32)] * 2,
    )(x, W_msg)


def _prep_c(edge_attr, W_msg, b_msg, D):
    E, DE = edge_attr.shape
    blk = 2000
    assert E % blk == 0

    def body(e_ref, w_ref, b_ref, o_ref):
        o_ref[...] = (
            jnp.dot(e_ref[...], w_ref[2 * D:2 * D + DE, :],
                    preferred_element_type=jnp.float32)
            + b_ref[...]
        )

    return pl.pallas_call(
        body,
        grid=(E // blk,),
        in_specs=[
            pl.BlockSpec((blk, DE), lambda i: (i, 0)),
            pl.BlockSpec(W_msg.shape, lambda i: (0, 0)),
            pl.BlockSpec((1, D), lambda i: (0, 0)),
        ],
        out_specs=pl.BlockSpec((blk, D), lambda i: (i, 0)),
        out_shape=jax.ShapeDtypeStruct((E, D), jnp.float32),
    )(edge_attr, W_msg, b_msg.reshape(1, D))


def _edge_sc(A, B, C, src, dst):
    N, D = A.shape
    E = C.shape[0]
    assert E % _NW == 0
    EW = E // _NW
    assert EW % 8 == 0
    K = 56
    FULL = (EW // K) // 2 * 2
    REM = EW - FULL * K
    assert REM % 8 == 0 and REM < 2 * K
    ROWS_T = (N // _NS) // 8 * 8
    TAIL = N - _NS * ROWS_T
    NZ = ROWS_T // K
    ZREM = ROWS_T - NZ * K
    assert ZREM % 8 == 0 and TAIL % 8 == 0

    mesh = plsc.VectorSubcoreMesh(core_axis_name="c", subcore_axis_name="s")
    dbl = lambda t: [t, t]
    scratch = [
        dbl(pltpu.VMEM((K,), jnp.int32)),
        dbl(pltpu.VMEM((K,), jnp.int32)),
        dbl(pltpu.VMEM((K, D), jnp.float32)),
        dbl(pltpu.VMEM((K, D), jnp.float32)),
        dbl(pltpu.VMEM((K, D), jnp.float32)),
        pltpu.VMEM((max(REM, 8),), jnp.int32),
        pltpu.VMEM((max(REM, 8),), jnp.int32),
        pltpu.VMEM_SHARED((N, D), jnp.float32),
        dbl(pltpu.SemaphoreType.DMA),
        dbl(pltpu.SemaphoreType.DMA),
        dbl(pltpu.SemaphoreType.DMA),
        dbl(pltpu.SemaphoreType.DMA),
    ]

    @functools.partial(
        pl.kernel,
        out_type=jax.ShapeDtypeStruct((_NC, N, D), jnp.float32),
        mesh=mesh,
        scratch_types=scratch,
    )
    def body(a_hbm, b_hbm, c_hbm, src_hbm, dst_hbm, out_hbm,
             srcv, dstv, av, bv, cv, srcr, dstr,
             agg, sem_a, sem_b, sem_c, sem_s):
        cid = lax.axis_index("c")
        sid = lax.axis_index("s")
        wid = cid * _NS + sid

        zero = jnp.zeros((_L,), jnp.float32)

        def zrow(r, carry):
            for jj in range(D // _L):
                av[0][r, pl.ds(jj * _L, _L)] = zero
            return carry

        lax.fori_loop(0, K, zrow, 0)

        def zslice(k, carry):
            pltpu.sync_copy(av[0], agg.at[pl.ds(sid * ROWS_T + k * K, K)])
            return carry

        lax.fori_loop(0, NZ, zslice, 0)
        if ZREM:
            pltpu.sync_copy(av[0].at[pl.ds(0, ZREM)],
                            agg.at[pl.ds(sid * ROWS_T + NZ * K, ZREM)])

        @pl.when(sid == 0)
        def _zero_tail():
            pltpu.sync_copy(av[0].at[pl.ds(0, TAIL)],
                            agg.at[pl.ds(_NS * ROWS_T, TAIL)])

        plsc.subcore_barrier()

        base = wid * EW

        def issue(j, p):
            off = base + j * K
            pltpu.sync_copy(src_hbm.at[pl.ds(off, K)], srcv[p])
            pltpu.sync_copy(dst_hbm.at[pl.ds(off, K)], dstv[p])
            pltpu.async_copy(a_hbm.at[srcv[p]], av[p], sem_a[p])
            pltpu.async_copy(b_hbm.at[dstv[p]], bv[p], sem_b[p])
            pltpu.async_copy(c_hbm.at[pl.ds(off, K)], cv[p], sem_c[p])

        def wait_gathers(j, p):
            off = base + j * K
            pltpu.make_async_copy(a_hbm.at[srcv[p]], av[p], sem_a[p]).wait()
            pltpu.make_async_copy(b_hbm.at[dstv[p]], bv[p], sem_b[p]).wait()
            pltpu.make_async_copy(c_hbm.at[pl.ds(off, K)], cv[p], sem_c[p]).wait()

        def compute(p):
            a_b, b_b, c_b = av[p], bv[p], cv[p]

            def crow(r, carry):
                for jj in range(D // _L):
                    sl = pl.ds(jj * _L, _L)
                    a_b[r, sl] = jnp.maximum(a_b[r, sl] + b_b[r, sl] + c_b[r, sl], 0.0)
                return carry

            lax.fori_loop(0, K, crow, 0)

        def scatter_issue(p):
            pltpu.async_copy(av[p], agg.at[dstv[p]], sem_s[p], add=True)

        def scatter_wait(p):
            pltpu.make_async_copy(av[p], agg.at[dstv[p]], sem_s[p]).wait()

        issue(0, 0)
        issue(1, 1)

        def pair(j2, carry):
            e = 2 * j2
            wait_gathers(e, 0)
            compute(0)
            scatter_issue(0)
            wait_gathers(e + 1, 1)
            compute(1)
            scatter_issue(1)

            @pl.when(e + 2 < FULL)
            def _next0():
                scatter_wait(0)
                issue(e + 2, 0)

            @pl.when(e + 3 < FULL)
            def _next1():
                scatter_wait(1)
                issue(e + 3, 1)

            return carry

        lax.fori_loop(0, FULL // 2, pair, 0)
        scatter_wait(0)
        scatter_wait(1)

        if REM:
            off = base + FULL * K
            pltpu.sync_copy(src_hbm.at[pl.ds(off, REM)], srcr)
            pltpu.sync_copy(dst_hbm.at[pl.ds(off, REM)], dstr)
            a_b = av[0].at[pl.ds(0, REM)]
            b_b = bv[0].at[pl.ds(0, REM)]
            c_b = cv[0].at[pl.ds(0, REM)]
            pltpu.async_copy(a_hbm.at[srcr], a_b, sem_a[0])
            pltpu.async_copy(b_hbm.at[dstr], b_b, sem_b[0])
            pltpu.async_copy(c_hbm.at[pl.ds(off, REM)], c_b, sem_c[0])
            pltpu.make_async_copy(a_hbm.at[srcr], a_b, sem_a[0]).wait()
            pltpu.make_async_copy(b_hbm.at[dstr], b_b, sem_b[0]).wait()
            pltpu.make_async_copy(c_hbm.at[pl.ds(off, REM)], c_b, sem_c[0]).wait()

            def rrow(r, carry):
                for jj in range(D // _L):
                    sl = pl.ds(jj * _L, _L)
                    av[0][r, sl] = jnp.maximum(
                        av[0][r, sl] + bv[0][r, sl] + cv[0][r, sl], 0.0)
                return carry

            lax.fori_loop(0, REM, rrow, 0)
            pltpu.sync_copy(a_b, agg.at[dstr], add=True)

        plsc.subcore_barrier()

        def orow(k, carry):
            r0 = sid * ROWS_T + k * K
            pltpu.sync_copy(agg.at[pl.ds(r0, K)], av[0])
            pltpu.sync_copy(av[0], out_hbm.at[cid, pl.ds(r0, K)])
            return carry

        lax.fori_loop(0, NZ, orow, 0)
        if ZREM:
            r0 = sid * ROWS_T + NZ * K
            pltpu.sync_copy(agg.at[pl.ds(r0, ZREM)], av[0].at[pl.ds(0, ZREM)])
            pltpu.sync_copy(av[0].at[pl.ds(0, ZREM)],
                            out_hbm.at[cid, pl.ds(r0, ZREM)])

        @pl.when(sid == 0)
        def _out_tail():
            r0 = _NS * ROWS_T
            pltpu.sync_copy(agg.at[pl.ds(r0, TAIL)], av[0].at[pl.ds(0, TAIL)])
            pltpu.sync_copy(av[0].at[pl.ds(0, TAIL)],
                            out_hbm.at[cid, pl.ds(r0, TAIL)])

    return body(A, B, C, src, dst)


def _update(x, agg2, W_upd, b_upd):
    N, D = x.shape
    blk = 2000
    assert N % blk == 0

    def body(x_ref, g_ref, w_ref, b_ref, o_ref):
        xb = x_ref[...]
        g = g_ref[0] + g_ref[1]
        h = (
            jnp.dot(xb, w_ref[0:D, :], preferred_element_type=jnp.float32)
            + jnp.dot(g, w_ref[D:2 * D, :], preferred_element_type=jnp.float32)
            + b_ref[...]
        )
        o_ref[...] = xb + jnp.maximum(h, 0.0)

    return pl.pallas_call(
        body,
        grid=(N // blk,),
        in_specs=[
            pl.BlockSpec((blk, D), lambda i: (i, 0)),
            pl.BlockSpec((2, blk, D), lambda i: (0, i, 0)),
            pl.BlockSpec(W_upd.shape, lambda i: (0, 0)),
            pl.BlockSpec((1, D), lambda i: (0, 0)),
        ],
        out_specs=pl.BlockSpec((blk, D), lambda i: (i, 0)),
        out_shape=jax.ShapeDtypeStruct((N, D), jnp.float32),
    )(x, agg2, W_upd, b_upd.reshape(1, D))


def kernel(x, edge_index, edge_attr, W_msg, b_msg, W_upd, b_upd):
    N, D = x.shape
    src = edge_index[0].astype(jnp.int32)
    dst = edge_index[1].astype(jnp.int32)
    A, B = _prep_ab(x, W_msg)
    C = _prep_c(edge_attr, W_msg, b_msg, D)
    agg2 = _edge_sc(A, B, C, src, dst)
    return _update(x, agg2, W_upd, b_upd)
